# Initial kernel scaffold; baseline (speedup 1.0000x reference)
import jax
import jax.numpy as jnp
from jax.experimental import pallas as pl


def kernel(h, edge_index):
    raise NotImplementedError("write your pallas kernel here")


# speedup vs baseline: 2.4367x; 2.4367x over previous
import functools

import jax
import jax.numpy as jnp
from jax import lax
from jax.experimental import pallas as pl
from jax.experimental.pallas import tpu as pltpu
from jax.experimental.pallas import tpu_sc as plsc

E = 320000
D = 128
N_WORKERS = 32
EPW = E // N_WORKERS
CHUNK = 80
N_CHUNKS = EPW // CHUNK
N_GROUPS = CHUNK // 16


def _sc_body(h_hbm, src_hbm, dst_hbm, out_hbm,
             idx_s, idx_d, rows_s, rows_d, out_v, pbuf, sem_s, sem_d):
    cid = lax.axis_index("c")
    sid = lax.axis_index("s")
    wid = sid * 2 + cid
    base = wid * EPW
    lane = lax.iota(jnp.int32, 16)

    def chunk_body(ci, carry):
        off = pl.multiple_of(base + ci * CHUNK, 8)
        pltpu.sync_copy(src_hbm.at[pl.ds(off, CHUNK)], idx_s)
        pltpu.sync_copy(dst_hbm.at[pl.ds(off, CHUNK)], idx_d)
        cp_s = pltpu.async_copy(h_hbm.at[idx_s], rows_s, sem_s)
        cp_d = pltpu.async_copy(h_hbm.at[idx_d], rows_d, sem_d)
        cp_s.wait()
        cp_d.wait()
        for g in range(N_GROUPS):
            for e in range(16):
                r = g * 16 + e
                p = rows_s[r, pl.ds(0, 16)] * rows_d[r, pl.ds(0, 16)]
                for k in range(1, D // 16):
                    p = p + (rows_s[r, pl.ds(k * 16, 16)]
                             * rows_d[r, pl.ds(k * 16, 16)])
                pbuf[pl.ds(e * 16, 16)] = p
            acc = jnp.zeros((16,), jnp.float32)
            scaled = lane * 16
            for l in range(16):
                col = plsc.load_gather(pbuf, [scaled + l])
                acc = acc + col
            out_v[pl.ds(g * 16, 16)] = acc
        pltpu.sync_copy(out_v, out_hbm.at[pl.ds(off, CHUNK)])
        return carry

    lax.fori_loop(0, N_CHUNKS, chunk_body, 0)


@functools.partial(jax.jit, donate_argnums=())
def _sc_call(h, src, dst):
    mesh = plsc.VectorSubcoreMesh(core_axis_name="c", subcore_axis_name="s")
    fn = pl.kernel(
        _sc_body,
        out_type=jax.ShapeDtypeStruct((E,), jnp.float32),
        mesh=mesh,
        compiler_params=pltpu.CompilerParams(needs_layout_passes=False),
        scratch_types=[
            pltpu.VMEM((CHUNK,), jnp.int32),
            pltpu.VMEM((CHUNK,), jnp.int32),
            pltpu.VMEM((CHUNK, D), jnp.float32),
            pltpu.VMEM((CHUNK, D), jnp.float32),
            pltpu.VMEM((CHUNK,), jnp.float32),
            pltpu.VMEM((256,), jnp.float32),
            pltpu.SemaphoreType.DMA,
            pltpu.SemaphoreType.DMA,
        ],
    )
    return fn(h, src, dst)


def kernel(h, edge_index):
    ei = edge_index.astype(jnp.int32)
    out = _sc_call(h, ei[0], ei[1])
    return out.reshape(E, 1)


# speedup vs baseline: 3.8077x; 1.5626x over previous
import functools

import jax
import jax.numpy as jnp
from jax import lax
from jax.experimental import pallas as pl
from jax.experimental.pallas import tpu as pltpu
from jax.experimental.pallas import tpu_sc as plsc

E = 320000
D = 128
N_WORKERS = 32
EPW = E // N_WORKERS
CHUNK = 80
N_CHUNKS = EPW // CHUNK
N_GROUPS = CHUNK // 16


def _sc_body(h_hbm, src_hbm, dst_hbm, out_hbm,
             idx_s, idx_d, rows_s0, rows_s1, rows_d0, rows_d1, out_all, pbuf,
             sem_s0, sem_s1, sem_d0, sem_d1):
    cid = lax.axis_index("c")
    sid = lax.axis_index("s")
    wid = sid * 2 + cid
    base = wid * EPW
    lane = lax.iota(jnp.int32, 16)
    rows_s = (rows_s0, rows_s1)
    rows_d = (rows_d0, rows_d1)
    sem_s = (sem_s0, sem_s1)
    sem_d = (sem_d0, sem_d1)

    pltpu.sync_copy(src_hbm.at[pl.ds(base, EPW)], idx_s)
    pltpu.sync_copy(dst_hbm.at[pl.ds(base, EPW)], idx_d)

    def _islice(ref, ci):
        return ref.at[pl.ds(pl.multiple_of(ci * CHUNK, 8), CHUNK)]

    def issue(ci, b):
        pltpu.async_copy(h_hbm.at[_islice(idx_s, ci)], rows_s[b], sem_s[b])
        pltpu.async_copy(h_hbm.at[_islice(idx_d, ci)], rows_d[b], sem_d[b])

    def wait(ci, b):
        pltpu.make_async_copy(
            h_hbm.at[_islice(idx_s, ci)], rows_s[b], sem_s[b]).wait()
        pltpu.make_async_copy(
            h_hbm.at[_islice(idx_d, ci)], rows_d[b], sem_d[b]).wait()

    def compute(ci, b):
        rs, rd = rows_s[b], rows_d[b]
        for g in range(N_GROUPS):
            for e in range(16):
                r = g * 16 + e
                p = rs[r, pl.ds(0, 16)] * rd[r, pl.ds(0, 16)]
                for k in range(1, D // 16):
                    p = p + rs[r, pl.ds(k * 16, 16)] * rd[r, pl.ds(k * 16, 16)]
                pbuf[pl.ds(e * 16, 16)] = p
            acc = jnp.zeros((16,), jnp.float32)
            scaled = lane * 16
            for l in range(16):
                acc = acc + plsc.load_gather(pbuf, [scaled + l])
            out_all[pl.ds(ci * CHUNK + g * 16, 16)] = acc

    issue(0, 0)

    def pair_body(i2, carry):
        for b in range(2):
            ci = i2 * 2 + b
            issue(ci + 1, 1 - b)
            wait(ci, b)
            compute(ci, b)
        return carry

    lax.fori_loop(0, (N_CHUNKS - 1) // 2, pair_body, 0)
    last = N_CHUNKS - 1
    wait(last, last % 2)
    compute(last, last % 2)

    pltpu.sync_copy(out_all, out_hbm.at[pl.ds(base, EPW)])


@jax.jit
def _sc_call(h, src, dst):
    mesh = plsc.VectorSubcoreMesh(core_axis_name="c", subcore_axis_name="s")
    fn = pl.kernel(
        _sc_body,
        out_type=jax.ShapeDtypeStruct((E,), jnp.float32),
        mesh=mesh,
        compiler_params=pltpu.CompilerParams(needs_layout_passes=False),
        scratch_types=[
            pltpu.VMEM((EPW,), jnp.int32),
            pltpu.VMEM((EPW,), jnp.int32),
            pltpu.VMEM((CHUNK, D), jnp.float32),
            pltpu.VMEM((CHUNK, D), jnp.float32),
            pltpu.VMEM((CHUNK, D), jnp.float32),
            pltpu.VMEM((CHUNK, D), jnp.float32),
            pltpu.VMEM((EPW,), jnp.float32),
            pltpu.VMEM((256,), jnp.float32),
            pltpu.SemaphoreType.DMA,
            pltpu.SemaphoreType.DMA,
            pltpu.SemaphoreType.DMA,
            pltpu.SemaphoreType.DMA,
        ],
    )
    return fn(h, src, dst)


def kernel(h, edge_index):
    ei = edge_index.astype(jnp.int32)
    src = ei[0]
    dst = ei[1]
    out = _sc_call(h, src, dst)
    return out.reshape(E, 1)


# speedup vs baseline: 5.9794x; 1.5703x over previous
import functools

import jax
import jax.numpy as jnp
from jax import lax
from jax.experimental import pallas as pl
from jax.experimental.pallas import tpu as pltpu
from jax.experimental.pallas import tpu_sc as plsc

E = 320000
D = 128
N_WORKERS = 32
EPW = E // N_WORKERS
CHUNK = 80
N_CHUNKS = EPW // CHUNK
N_GROUPS = CHUNK // 16


def _sc_body(h_hbm, src_hbm, dst_hbm, out_hbm,
             idx_s, idx_d, rows_s0, rows_s1, rows_d0, rows_d1, out_all, pbuf,
             sem_s0, sem_s1, sem_d0, sem_d1):
    cid = lax.axis_index("c")
    sid = lax.axis_index("s")
    wid = sid * 2 + cid
    base = wid * EPW
    lane = lax.iota(jnp.int32, 16)
    rows_s = (rows_s0, rows_s1)
    rows_d = (rows_d0, rows_d1)
    sem_s = (sem_s0, sem_s1)
    sem_d = (sem_d0, sem_d1)

    pltpu.sync_copy(src_hbm.at[pl.ds(base, EPW)], idx_s)
    pltpu.sync_copy(dst_hbm.at[pl.ds(base, EPW)], idx_d)

    def _islice(ref, ci):
        return ref.at[pl.ds(pl.multiple_of(ci * CHUNK, 8), CHUNK)]

    def issue(ci, b):
        pltpu.async_copy(h_hbm.at[_islice(idx_s, ci)], rows_s[b], sem_s[b])
        pltpu.async_copy(h_hbm.at[_islice(idx_d, ci)], rows_d[b], sem_d[b])

    def wait(ci, b):
        pltpu.make_async_copy(
            h_hbm.at[_islice(idx_s, ci)], rows_s[b], sem_s[b]).wait()
        pltpu.make_async_copy(
            h_hbm.at[_islice(idx_d, ci)], rows_d[b], sem_d[b]).wait()

    def compute(ci, b):
        rs, rd = rows_s[b], rows_d[b]
        for g in range(N_GROUPS):
            for e in range(16):
                r = g * 16 + e
                p = jnp.zeros((16,), jnp.float32)
                for k in range(D // 32):
                    sv = plsc.bitcast(rs[r, pl.ds(k * 16, 16)], jnp.bfloat16)
                    dv = plsc.bitcast(rd[r, pl.ds(k * 16, 16)], jnp.bfloat16)
                    s1, s2 = plsc.unpack(sv, format=plsc.PackFormat.INTERLEAVED)
                    d1, d2 = plsc.unpack(dv, format=plsc.PackFormat.INTERLEAVED)
                    p = p + s1 * d1 + s2 * d2
                pbuf[pl.ds(e * 16, 16)] = p
            acc = jnp.zeros((16,), jnp.float32)
            scaled = lane * 16
            for l in range(16):
                acc = acc + plsc.load_gather(pbuf, [scaled + l])
            out_all[pl.ds(ci * CHUNK + g * 16, 16)] = acc

    issue(0, 0)

    def pair_body(i2, carry):
        for b in range(2):
            ci = i2 * 2 + b
            issue(ci + 1, 1 - b)
            wait(ci, b)
            compute(ci, b)
        return carry

    lax.fori_loop(0, (N_CHUNKS - 1) // 2, pair_body, 0)
    last = N_CHUNKS - 1
    wait(last, last % 2)
    compute(last, last % 2)

    pltpu.sync_copy(out_all, out_hbm.at[pl.ds(base, EPW)])


@jax.jit
def _sc_call(h, src, dst):
    mesh = plsc.VectorSubcoreMesh(core_axis_name="c", subcore_axis_name="s")
    fn = pl.kernel(
        _sc_body,
        out_type=jax.ShapeDtypeStruct((E,), jnp.float32),
        mesh=mesh,
        compiler_params=pltpu.CompilerParams(
            needs_layout_passes=False, use_tc_tiling_on_sc=False),
        scratch_types=[
            pltpu.VMEM((EPW,), jnp.int32),
            pltpu.VMEM((EPW,), jnp.int32),
            pltpu.VMEM((CHUNK, D // 2), jnp.int32),
            pltpu.VMEM((CHUNK, D // 2), jnp.int32),
            pltpu.VMEM((CHUNK, D // 2), jnp.int32),
            pltpu.VMEM((CHUNK, D // 2), jnp.int32),
            pltpu.VMEM((EPW,), jnp.float32),
            pltpu.VMEM((256,), jnp.float32),
            pltpu.SemaphoreType.DMA,
            pltpu.SemaphoreType.DMA,
            pltpu.SemaphoreType.DMA,
            pltpu.SemaphoreType.DMA,
        ],
    )
    return fn(h, src, dst)


def kernel(h, edge_index):
    ei = edge_index.astype(jnp.int32)
    src = ei[0]
    dst = ei[1]
    h_packed = jax.lax.bitcast_convert_type(
        h.astype(jnp.bfloat16).reshape(h.shape[0], D // 2, 2), jnp.int32)
    out = _sc_call(h_packed, src, dst)
    return out.reshape(E, 1)


# speedup vs baseline: 6.0825x; 1.0172x over previous
import functools

import jax
import jax.numpy as jnp
from jax import lax
from jax.experimental import pallas as pl
from jax.experimental.pallas import tpu as pltpu
from jax.experimental.pallas import tpu_sc as plsc

E = 320000
D = 128
N_WORKERS = 32
EPW = E // N_WORKERS
CHUNK = 80
N_CHUNKS = EPW // CHUNK
N_GROUPS = CHUNK // 16


def _sc_body(h_hbm, src_hbm, dst_hbm, out_hbm,
             h_sp, idx_s, idx_d, rows_s0, rows_s1, rows_d0, rows_d1,
             out_all, pbuf, sem_s0, sem_s1, sem_d0, sem_d1):
    cid = lax.axis_index("c")
    sid = lax.axis_index("s")
    wid = sid * 2 + cid
    base = wid * EPW
    lane = lax.iota(jnp.int32, 16)
    rows_s = (rows_s0, rows_s1)
    rows_d = (rows_d0, rows_d1)
    sem_s = (sem_s0, sem_s1)
    sem_d = (sem_d0, sem_d1)

    @pl.when(sid == 0)
    def _stage_table():
        pltpu.sync_copy(h_hbm, h_sp)

    pltpu.sync_copy(src_hbm.at[pl.ds(base, EPW)], idx_s)
    pltpu.sync_copy(dst_hbm.at[pl.ds(base, EPW)], idx_d)
    plsc.subcore_barrier()

    def _islice(ref, ci):
        return ref.at[pl.ds(pl.multiple_of(ci * CHUNK, 8), CHUNK)]

    def issue(ci, b):
        pltpu.async_copy(h_sp.at[_islice(idx_s, ci)], rows_s[b], sem_s[b])
        pltpu.async_copy(h_sp.at[_islice(idx_d, ci)], rows_d[b], sem_d[b])

    def wait(ci, b):
        pltpu.make_async_copy(
            h_sp.at[_islice(idx_s, ci)], rows_s[b], sem_s[b]).wait()
        pltpu.make_async_copy(
            h_sp.at[_islice(idx_d, ci)], rows_d[b], sem_d[b]).wait()

    def compute(ci, b):
        rs, rd = rows_s[b], rows_d[b]
        for g in range(N_GROUPS):
            for e in range(16):
                r = g * 16 + e
                p = jnp.zeros((16,), jnp.float32)
                for k in range(D // 32):
                    sv = plsc.bitcast(rs[r, pl.ds(k * 16, 16)], jnp.bfloat16)
                    dv = plsc.bitcast(rd[r, pl.ds(k * 16, 16)], jnp.bfloat16)
                    s1, s2 = plsc.unpack(sv, format=plsc.PackFormat.INTERLEAVED)
                    d1, d2 = plsc.unpack(dv, format=plsc.PackFormat.INTERLEAVED)
                    p = p + s1 * d1 + s2 * d2
                pbuf[pl.ds(e * 16, 16)] = p
            acc = jnp.zeros((16,), jnp.float32)
            scaled = lane * 16
            for l in range(16):
                acc = acc + plsc.load_gather(pbuf, [scaled + l])
            out_all[pl.ds(ci * CHUNK + g * 16, 16)] = acc

    issue(0, 0)

    def pair_body(i2, carry):
        for b in range(2):
            ci = i2 * 2 + b
            issue(ci + 1, 1 - b)
            wait(ci, b)
            compute(ci, b)
        return carry

    lax.fori_loop(0, (N_CHUNKS - 1) // 2, pair_body, 0)
    last = N_CHUNKS - 1
    wait(last, last % 2)
    compute(last, last % 2)

    pltpu.sync_copy(out_all, out_hbm.at[pl.ds(base, EPW)])


@jax.jit
def _sc_call(h, src, dst):
    mesh = plsc.VectorSubcoreMesh(core_axis_name="c", subcore_axis_name="s")
    fn = pl.kernel(
        _sc_body,
        out_type=jax.ShapeDtypeStruct((E,), jnp.float32),
        mesh=mesh,
        compiler_params=pltpu.CompilerParams(
            needs_layout_passes=False, use_tc_tiling_on_sc=False),
        scratch_types=[
            pltpu.VMEM_SHARED((10000, D // 2), jnp.int32),
            pltpu.VMEM((EPW,), jnp.int32),
            pltpu.VMEM((EPW,), jnp.int32),
            pltpu.VMEM((CHUNK, D // 2), jnp.int32),
            pltpu.VMEM((CHUNK, D // 2), jnp.int32),
            pltpu.VMEM((CHUNK, D // 2), jnp.int32),
            pltpu.VMEM((CHUNK, D // 2), jnp.int32),
            pltpu.VMEM((EPW,), jnp.float32),
            pltpu.VMEM((256,), jnp.float32),
            pltpu.SemaphoreType.DMA,
            pltpu.SemaphoreType.DMA,
            pltpu.SemaphoreType.DMA,
            pltpu.SemaphoreType.DMA,
        ],
    )
    return fn(h, src, dst)


def kernel(h, edge_index):
    ei = edge_index.astype(jnp.int32)
    src = ei[0]
    dst = ei[1]
    h_packed = jax.lax.bitcast_convert_type(
        h.astype(jnp.bfloat16).reshape(h.shape[0], D // 2, 2), jnp.int32)
    out = _sc_call(h_packed, src, dst)
    return out.reshape(E, 1)


# speedup vs baseline: 9.1601x; 1.5060x over previous
import functools

import jax
import jax.numpy as jnp
from jax import lax
from jax.experimental import pallas as pl
from jax.experimental.pallas import tpu as pltpu
from jax.experimental.pallas import tpu_sc as plsc

E = 320000
N = 10000
D = 128
W = D // 2
N_FGROUPS = 4
GROUP_TILES = 8
WPT = W // GROUP_TILES
EPG = E // N_FGROUPS
CHUNK = 1600
PHASE_E = (48000, 32000)
PHASE_C = (0, PHASE_E[0] // CHUNK)
PHASE_N = (PHASE_E[0] // CHUNK, PHASE_E[1] // CHUNK)
RED_SUB = 2000


def _sc_body(ht_hbm, src_hbm, dst_hbm, out_hbm,
             parts_sp, table_v, idx_s0, idx_s1, idx_d0, idx_d1,
             part0, part1, red_v, out_all,
             sem_i0, sem_i1, sem_p0, sem_p1):
    cid = lax.axis_index("c")
    sid = lax.axis_index("s")
    g = sid // GROUP_TILES
    r = sid % GROUP_TILES
    p = cid * 2 + g
    ebase = p * EPG
    idx_s = (idx_s0, idx_s1)
    idx_d = (idx_d0, idx_d1)
    part = (part0, part1)
    sem_i = (sem_i0, sem_i1)
    sem_p = (sem_p0, sem_p1)

    pltpu.sync_copy(
        ht_hbm.at[pl.ds(pl.multiple_of(r * WPT * N, 8), WPT * N)], table_v)

    def issue_idx(gi, b):
        off = pl.multiple_of(ebase + gi * CHUNK, 8)
        pltpu.async_copy(src_hbm.at[pl.ds(off, CHUNK)], idx_s[b], sem_i[b])
        pltpu.async_copy(dst_hbm.at[pl.ds(off, CHUNK)], idx_d[b], sem_i[b])

    def wait_idx(gi, b):
        off = pl.multiple_of(ebase + gi * CHUNK, 8)
        pltpu.make_async_copy(
            src_hbm.at[pl.ds(off, CHUNK)], idx_s[b], sem_i[b]).wait()
        pltpu.make_async_copy(
            dst_hbm.at[pl.ds(off, CHUNK)], idx_d[b], sem_i[b]).wait()

    def part_dst(ci):
        return parts_sp.at[
            g, r, pl.ds(pl.multiple_of(ci * (CHUNK // 2), 8), CHUNK // 2)]

    lane2 = lax.iota(jnp.int32, 16) * 2

    def dot16(ks, kd):
        acc = jnp.zeros((16,), jnp.float32)
        for w in range(WPT):
            sw = plsc.load_gather(table_v, [ks + (w * N)])
            dw = plsc.load_gather(table_v, [kd + (w * N)])
            sb = plsc.bitcast(sw, jnp.bfloat16)
            db = plsc.bitcast(dw, jnp.bfloat16)
            s1, s2 = plsc.unpack(sb, format=plsc.PackFormat.INTERLEAVED)
            d1, d2 = plsc.unpack(db, format=plsc.PackFormat.INTERLEAVED)
            acc = acc + s1 * d1 + s2 * d2
        return acc

    def compute(ci, b):
        isv, idv, pv = idx_s[b], idx_d[b], part[b]

        def group_body(j, carry):
            base32 = j * 32
            ks_a = plsc.load_gather(isv, [lane2 + base32])
            ks_b = plsc.load_gather(isv, [lane2 + (base32 + 1)])
            kd_a = plsc.load_gather(idv, [lane2 + base32])
            kd_b = plsc.load_gather(idv, [lane2 + (base32 + 1)])
            acc_a = dot16(ks_a, kd_a)
            acc_b = dot16(ks_b, kd_b)
            packed = plsc.bitcast(
                plsc.pack(acc_a, acc_b, format=plsc.PackFormat.INTERLEAVED),
                jnp.int32)
            pv[pl.ds(pl.multiple_of(j * 16, 16), 16)] = packed
            return carry

        lax.fori_loop(0, CHUNK // 32, group_body, 0)
        pltpu.async_copy(pv, part_dst(ci), sem_p[b])

    def wait_part(ci, b):
        pltpu.make_async_copy(part[b], part_dst(ci), sem_p[b]).wait()

    for ph in range(2):
        pc0, nc = PHASE_C[ph], PHASE_N[ph]
        ept_ph = PHASE_E[ph] // GROUP_TILES
        n_red = ept_ph // RED_SUB

        issue_idx(pc0, 0)

        def pair_body(i2, carry):
            for b in range(2):
                ci = i2 * 2 + b
                issue_idx(pc0 + ci + 1, 1 - b)
                wait_idx(pc0 + ci, b)

                @pl.when(ci >= 2)
                def _drain():
                    wait_part(ci - 2, b)

                compute(ci, b)
            return carry

        lax.fori_loop(0, (nc - 2) // 2, pair_body, 0)
        c0, c1 = nc - 2, nc - 1
        issue_idx(pc0 + c1, c1 % 2)
        wait_idx(pc0 + c0, c0 % 2)
        wait_part(c0 - 2, c0 % 2)
        compute(c0, c0 % 2)
        wait_idx(pc0 + c1, c1 % 2)
        wait_part(c1 - 2, c1 % 2)
        compute(c1, c1 % 2)
        wait_part(c0, c0 % 2)
        wait_part(c1, c1 % 2)

        plsc.subcore_barrier()

        for sub in range(n_red):
            soff = pl.multiple_of((r * ept_ph + sub * RED_SUB) // 2, 8)
            pltpu.sync_copy(
                parts_sp.at[g, :, pl.ds(soff, RED_SUB // 2)], red_v)

            def red_body(j, carry):
                base16 = jnp.minimum(j * 16, RED_SUB // 2 - 16)
                acc_a = jnp.zeros((16,), jnp.float32)
                acc_b = jnp.zeros((16,), jnp.float32)
                for t in range(GROUP_TILES):
                    pb = plsc.bitcast(
                        red_v[t, pl.ds(base16, 16)], jnp.bfloat16)
                    a, b2 = plsc.unpack(
                        pb, format=plsc.PackFormat.INTERLEAVED)
                    acc_a = acc_a + a
                    acc_b = acc_b + b2
                obase = sub * RED_SUB + base16 * 2
                plsc.store_scatter(out_all, [lane2 + obase], acc_a)
                plsc.store_scatter(out_all, [lane2 + (obase + 1)], acc_b)
                return carry

            lax.fori_loop(0, (RED_SUB // 2 + 15) // 16, red_body, 0)

        obase = pl.multiple_of(p * EPG + pc0 * CHUNK + r * ept_ph, 8)
        pltpu.sync_copy(
            out_all.at[pl.ds(0, ept_ph)], out_hbm.at[pl.ds(obase, ept_ph)])

        plsc.subcore_barrier()


@jax.jit
def _sc_call(ht, src, dst):
    mesh = plsc.VectorSubcoreMesh(core_axis_name="c", subcore_axis_name="s")
    fn = pl.kernel(
        _sc_body,
        out_type=jax.ShapeDtypeStruct((E,), jnp.float32),
        mesh=mesh,
        compiler_params=pltpu.CompilerParams(
            needs_layout_passes=False, use_tc_tiling_on_sc=False),
        scratch_types=[
            pltpu.VMEM_SHARED((2, GROUP_TILES, PHASE_E[0] // 2), jnp.int32),
            pltpu.VMEM((WPT * N,), jnp.int32),
            pltpu.VMEM((CHUNK,), jnp.int32),
            pltpu.VMEM((CHUNK,), jnp.int32),
            pltpu.VMEM((CHUNK,), jnp.int32),
            pltpu.VMEM((CHUNK,), jnp.int32),
            pltpu.VMEM((CHUNK // 2,), jnp.int32),
            pltpu.VMEM((CHUNK // 2,), jnp.int32),
            pltpu.VMEM((GROUP_TILES, RED_SUB // 2), jnp.int32),
            pltpu.VMEM((PHASE_E[0] // GROUP_TILES,), jnp.float32),
            pltpu.SemaphoreType.DMA,
            pltpu.SemaphoreType.DMA,
            pltpu.SemaphoreType.DMA,
            pltpu.SemaphoreType.DMA,
        ],
    )
    return fn(ht, src, dst)


def kernel(h, edge_index):
    ei = edge_index.astype(jnp.int32)
    h_packed = jax.lax.bitcast_convert_type(
        h.astype(jnp.bfloat16).reshape(N, W, 2), jnp.int32)
    ht = h_packed.T.reshape(W * N)
    out = _sc_call(ht, ei[0], ei[1])
    return out.reshape(E, 1)


# speedup vs baseline: 9.6693x; 1.0556x over previous
import functools

import jax
import jax.numpy as jnp
from jax import lax
from jax.experimental import pallas as pl
from jax.experimental.pallas import tpu as pltpu
from jax.experimental.pallas import tpu_sc as plsc

E = 320000
N = 10000
D = 128
W = D // 2
N_FGROUPS = 4
GROUP_TILES = 8
WPT = W // GROUP_TILES
EPG = E // N_FGROUPS
CHUNK = 1600
PHASE_E = (48000, 32000)
PHASE_C = (0, PHASE_E[0] // CHUNK)
PHASE_N = (PHASE_E[0] // CHUNK, PHASE_E[1] // CHUNK)
RED_SUB = 2000


def _sc_body(ht_hbm, src_hbm, dst_hbm, out_hbm,
             parts_sp, table_v, idx_s0, idx_s1, idx_d0, idx_d1,
             part0, part1, red_v, out_all,
             sem_i0, sem_i1, sem_p0, sem_p1):
    cid = lax.axis_index("c")
    sid = lax.axis_index("s")
    g = sid // GROUP_TILES
    r = sid % GROUP_TILES
    p = cid * 2 + g
    ebase = p * EPG
    idx_s = (idx_s0, idx_s1)
    idx_d = (idx_d0, idx_d1)
    part = (part0, part1)
    sem_i = (sem_i0, sem_i1)
    sem_p = (sem_p0, sem_p1)

    pltpu.sync_copy(
        ht_hbm.at[pl.ds(pl.multiple_of(r * WPT * N, 8), WPT * N)], table_v)

    def issue_idx(gi, b):
        off = pl.multiple_of(ebase + gi * CHUNK, 8)
        pltpu.async_copy(src_hbm.at[pl.ds(off, CHUNK)], idx_s[b], sem_i[b])
        pltpu.async_copy(dst_hbm.at[pl.ds(off, CHUNK)], idx_d[b], sem_i[b])

    def wait_idx(gi, b):
        off = pl.multiple_of(ebase + gi * CHUNK, 8)
        pltpu.make_async_copy(
            src_hbm.at[pl.ds(off, CHUNK)], idx_s[b], sem_i[b]).wait()
        pltpu.make_async_copy(
            dst_hbm.at[pl.ds(off, CHUNK)], idx_d[b], sem_i[b]).wait()

    def part_dst(ci):
        return parts_sp.at[
            g, r, pl.ds(pl.multiple_of(ci * (CHUNK // 2), 8), CHUNK // 2)]

    lane2 = lax.iota(jnp.int32, 16) * 2

    def dot16(ks, kd):
        acc = jnp.zeros((32,), jnp.bfloat16)
        for w in range(WPT):
            sw = plsc.load_gather(table_v, [ks + (w * N)])
            dw = plsc.load_gather(table_v, [kd + (w * N)])
            sb = plsc.bitcast(sw, jnp.bfloat16)
            db = plsc.bitcast(dw, jnp.bfloat16)
            acc = acc + sb * db
        a1, a2 = plsc.unpack(acc, format=plsc.PackFormat.INTERLEAVED)
        return a1 + a2

    def compute(ci, b):
        isv, idv, pv = idx_s[b], idx_d[b], part[b]

        def group_body(j, carry):
            base32 = j * 32
            ks_a = plsc.load_gather(isv, [lane2 + base32])
            ks_b = plsc.load_gather(isv, [lane2 + (base32 + 1)])
            kd_a = plsc.load_gather(idv, [lane2 + base32])
            kd_b = plsc.load_gather(idv, [lane2 + (base32 + 1)])
            acc_a = dot16(ks_a, kd_a)
            acc_b = dot16(ks_b, kd_b)
            packed = plsc.bitcast(
                plsc.pack(acc_a, acc_b, format=plsc.PackFormat.INTERLEAVED),
                jnp.int32)
            pv[pl.ds(pl.multiple_of(j * 16, 16), 16)] = packed
            return carry

        lax.fori_loop(0, CHUNK // 32, group_body, 0)
        pltpu.async_copy(pv, part_dst(ci), sem_p[b])

    def wait_part(ci, b):
        pltpu.make_async_copy(part[b], part_dst(ci), sem_p[b]).wait()

    for ph in range(2):
        pc0, nc = PHASE_C[ph], PHASE_N[ph]
        ept_ph = PHASE_E[ph] // GROUP_TILES
        n_red = ept_ph // RED_SUB

        issue_idx(pc0, 0)

        def pair_body(i2, carry):
            for b in range(2):
                ci = i2 * 2 + b
                issue_idx(pc0 + ci + 1, 1 - b)
                wait_idx(pc0 + ci, b)

                @pl.when(ci >= 2)
                def _drain():
                    wait_part(ci - 2, b)

                compute(ci, b)
            return carry

        lax.fori_loop(0, (nc - 2) // 2, pair_body, 0)
        c0, c1 = nc - 2, nc - 1
        issue_idx(pc0 + c1, c1 % 2)
        wait_idx(pc0 + c0, c0 % 2)
        wait_part(c0 - 2, c0 % 2)
        compute(c0, c0 % 2)
        wait_idx(pc0 + c1, c1 % 2)
        wait_part(c1 - 2, c1 % 2)
        compute(c1, c1 % 2)
        wait_part(c0, c0 % 2)
        wait_part(c1, c1 % 2)

        plsc.subcore_barrier()

        for sub in range(n_red):
            soff = pl.multiple_of((r * ept_ph + sub * RED_SUB) // 2, 8)
            pltpu.sync_copy(
                parts_sp.at[g, :, pl.ds(soff, RED_SUB // 2)], red_v)

            def red_body(j, carry):
                base16 = jnp.minimum(j * 16, RED_SUB // 2 - 16)
                acc_a = jnp.zeros((16,), jnp.float32)
                acc_b = jnp.zeros((16,), jnp.float32)
                for t in range(GROUP_TILES):
                    pb = plsc.bitcast(
                        red_v[t, pl.ds(base16, 16)], jnp.bfloat16)
                    a, b2 = plsc.unpack(
                        pb, format=plsc.PackFormat.INTERLEAVED)
                    acc_a = acc_a + a
                    acc_b = acc_b + b2
                obase = sub * RED_SUB + base16 * 2
                plsc.store_scatter(out_all, [lane2 + obase], acc_a)
                plsc.store_scatter(out_all, [lane2 + (obase + 1)], acc_b)
                return carry

            lax.fori_loop(0, (RED_SUB // 2 + 15) // 16, red_body, 0)

        obase = pl.multiple_of(p * EPG + pc0 * CHUNK + r * ept_ph, 8)
        pltpu.sync_copy(
            out_all.at[pl.ds(0, ept_ph)], out_hbm.at[pl.ds(obase, ept_ph)])

        plsc.subcore_barrier()


@jax.jit
def _sc_call(ht, src, dst):
    mesh = plsc.VectorSubcoreMesh(core_axis_name="c", subcore_axis_name="s")
    fn = pl.kernel(
        _sc_body,
        out_type=jax.ShapeDtypeStruct((E,), jnp.float32),
        mesh=mesh,
        compiler_params=pltpu.CompilerParams(
            needs_layout_passes=False, use_tc_tiling_on_sc=False),
        scratch_types=[
            pltpu.VMEM_SHARED((2, GROUP_TILES, PHASE_E[0] // 2), jnp.int32),
            pltpu.VMEM((WPT * N,), jnp.int32),
            pltpu.VMEM((CHUNK,), jnp.int32),
            pltpu.VMEM((CHUNK,), jnp.int32),
            pltpu.VMEM((CHUNK,), jnp.int32),
            pltpu.VMEM((CHUNK,), jnp.int32),
            pltpu.VMEM((CHUNK // 2,), jnp.int32),
            pltpu.VMEM((CHUNK // 2,), jnp.int32),
            pltpu.VMEM((GROUP_TILES, RED_SUB // 2), jnp.int32),
            pltpu.VMEM((PHASE_E[0] // GROUP_TILES,), jnp.float32),
            pltpu.SemaphoreType.DMA,
            pltpu.SemaphoreType.DMA,
            pltpu.SemaphoreType.DMA,
            pltpu.SemaphoreType.DMA,
        ],
    )
    return fn(ht, src, dst)


def kernel(h, edge_index):
    ei = edge_index.astype(jnp.int32)
    h_packed = jax.lax.bitcast_convert_type(
        h.astype(jnp.bfloat16).reshape(N, W, 2), jnp.int32)
    ht = h_packed.T.reshape(W * N)
    out = _sc_call(ht, ei[0], ei[1])
    return out.reshape(E, 1)
